# baseline (speedup 1.0000x reference)
import functools
import sys

import jax
import jax.numpy as jnp
from jax import lax
from jax.experimental import pallas as pl
from jax.experimental.pallas import tpu as pltpu
from jax.experimental.pallas import tpu_sc as plsc

_EPS = sys.float_info.epsilon
_V = 32768
_K = 3
_L = 16
_B = 8
_BIG_I32 = 2**30
_NEG = float("-inf")

_NC = 2
_NS = 16
_LANES = 16
_NW = _NC * _NS
_ROWS_PER_W = (_B * _L) // _NW
_NSETS = 8
_SET_ELEMS = _V // _NSETS
_SET_CHUNKS = _SET_ELEMS // _LANES
_UNROLL = 8
_NCAND = _NSETS * 0 + 3 * _LANES


def _top3_update(st, v, iv):
    m1, m2, m3, i1, i2, i3 = st
    c1 = v > m1
    nm1 = jnp.where(c1, v, m1)
    dv = jnp.where(c1, m1, v)
    ni1 = jnp.where(c1, iv, i1)
    di = jnp.where(c1, i1, iv)
    c2 = dv > m2
    nm2 = jnp.where(c2, dv, m2)
    d2v = jnp.where(c2, m2, dv)
    ni2 = jnp.where(c2, di, i2)
    d2i = jnp.where(c2, i2, di)
    c3 = d2v > m3
    nm3 = jnp.where(c3, d2v, m3)
    ni3 = jnp.where(c3, d2i, i3)
    return (nm1, nm2, nm3, ni1, ni2, ni3)


def _sc_body(x_hbm, outf_hbm, outi_hbm, buf, stage_f, stage_i, sem_a, sem_b):
    wid = lax.axis_index("s") * _NC + lax.axis_index("c")
    base_row = wid * _ROWS_PER_W
    lane = lax.iota(jnp.int32, _LANES)
    sems = [sem_a, sem_b]
    copies = [None, None]
    copies[0] = pltpu.async_copy(x_hbm.at[base_row], buf.at[0], sems[0])
    for j in range(_ROWS_PER_W):
        cur = j % 2
        nxt = (j + 1) % 2
        if j + 1 < _ROWS_PER_W:
            copies[nxt] = pltpu.async_copy(
                x_hbm.at[base_row + j + 1], buf.at[nxt], sems[nxt]
            )
        copies[cur].wait()
        bufj = buf.at[cur]

        neg = jnp.full((_LANES,), _NEG, jnp.float32)
        zero_i = jnp.zeros((_LANES,), jnp.int32)
        init = tuple((neg, neg, neg, zero_i, zero_i, zero_i) for _ in range(_NSETS))

        def sweep(i, sts, bufj=bufj, lane=lane):
            out = []
            for s in range(_NSETS):
                st = sts[s]
                for u in range(_UNROLL):
                    off = s * _SET_ELEMS + (i * _UNROLL + u) * _LANES
                    v = bufj[pl.ds(off, _LANES)]
                    iv = lane + off
                    st = _top3_update(st, v, iv)
                out.append(st)
            return tuple(out)

        sts = lax.fori_loop(0, _SET_CHUNKS // _UNROLL, sweep, init, unroll=False)

        st = sts[0]
        for s in range(1, _NSETS):
            ms1, ms2, ms3, is1, is2, is3 = sts[s]
            st = _top3_update(st, ms1, is1)
            st = _top3_update(st, ms2, is2)
            st = _top3_update(st, ms3, is3)
        m1, m2, m3, i1, i2, i3 = st

        stage_f[pl.ds(0, _LANES)] = m1
        stage_f[pl.ds(_LANES, _LANES)] = m2
        stage_f[pl.ds(2 * _LANES, _LANES)] = m3
        stage_i[pl.ds(0, _LANES)] = i1
        stage_i[pl.ds(_LANES, _LANES)] = i2
        stage_i[pl.ds(2 * _LANES, _LANES)] = i3
        r = base_row + j
        out_r = (r & (_L - 1)) * _B + (r >> 4)
        pltpu.sync_copy(stage_f, outf_hbm.at[out_r])
        pltpu.sync_copy(stage_i, outi_hbm.at[out_r])


@functools.cache
def _make_sc_rows():
    return functools.partial(
        pl.kernel,
        out_type=[
            jax.ShapeDtypeStruct((_B * _L, 3 * _LANES), jnp.float32),
            jax.ShapeDtypeStruct((_B * _L, 3 * _LANES), jnp.int32),
        ],
        mesh=plsc.VectorSubcoreMesh(
            core_axis_name="c", subcore_axis_name="s", num_cores=_NC, num_subcores=_NS
        ),
        scratch_types=[
            pltpu.VMEM((2, _V), jnp.float32),
            pltpu.VMEM((3 * _LANES,), jnp.float32),
            pltpu.VMEM((3 * _LANES,), jnp.int32),
            pltpu.SemaphoreType.DMA,
            pltpu.SemaphoreType.DMA,
        ],
    )(_sc_body)


def _sc_rows(rows):
    return _make_sc_rows()(rows)


def _z_body(x_ref, z_ref):
    x = x_ref[...]
    z = jnp.sum(jnp.exp(x), axis=1, keepdims=True)
    z_ref[...] = jnp.broadcast_to(z, (_B, 8))


def _beam_body(vf_ref, vi_ref, z_ref, seq_ref, sc_ref):
    vals = vf_ref[...]
    idx = vi_ref[...]
    lps, tks = [], []
    cur = vals
    for _ in range(_K):
        mj = jnp.max(cur, axis=1, keepdims=True)
        fj = jnp.min(jnp.where(cur == mj, idx, _BIG_I32), axis=1, keepdims=True)
        lps.append(mj)
        tks.append(fj)
        cur = jnp.where(idx == fj, _NEG, cur)
    v3_all = jnp.concatenate(lps, axis=1)
    tok_all = jnp.concatenate(tks, axis=1)
    lp_all = jnp.log(jnp.exp(v3_all) / z_ref[...] + _EPS)

    def top3(t):
        return (
            lp_all[t * _B : (t + 1) * _B, :],
            tok_all[t * _B : (t + 1) * _B, :],
        )

    scores, tok0 = top3(0)
    col = jax.lax.broadcasted_iota(jnp.int32, (_B, _L), 1)
    blocks = [jnp.where(col == 0, tok0[:, k : k + 1], 0) for k in range(_K)]
    for t in range(1, _L):
        lp, ix = top3(t)
        cand = jnp.concatenate(
            [scores[:, k : k + 1] + lp for k in range(_K)], axis=1
        )
        flat = jnp.concatenate([ix + k * _V for k in range(_K)], axis=1)
        rank = jnp.zeros((_B, 3 * _K), jnp.int32)
        for s in range(1, 3 * _K):
            rv = jnp.roll(cand, s, axis=1)
            rf = jnp.roll(flat, s, axis=1)
            beat = (rv > cand) | ((rv == cand) & (rf < flat))
            rank = rank + beat.astype(jnp.int32)
        ss, ff = [], []
        for r in range(_K):
            m = rank == r
            ss.append(jnp.sum(jnp.where(m, cand, 0.0), axis=1, keepdims=True))
            ff.append(jnp.sum(jnp.where(m, flat, 0), axis=1, keepdims=True))
        scores = jnp.concatenate(ss, axis=1)
        sel = jnp.concatenate(ff, axis=1)
        parent = sel >> 15
        token = sel & (_V - 1)
        nb = []
        for k in range(_K):
            pk = parent[:, k : k + 1]
            blk = jnp.where(pk == 0, blocks[0], jnp.where(pk == 1, blocks[1], blocks[2]))
            blk = jnp.where(col == t, token[:, k : k + 1], blk)
            nb.append(blk)
        blocks = nb
    seq_ref[...] = jnp.concatenate(blocks, axis=1)
    sc_ref[...] = jnp.concatenate(
        [scores, jnp.zeros((_B, 8 - _K), jnp.float32)], axis=1
    )


def kernel(logits):
    rows = logits.reshape(_B * _L, _V)
    vf_t, vi_t = _sc_rows(rows)
    zcols = pl.pallas_call(
        _z_body,
        grid=(_B * _L // 8,),
        in_specs=[pl.BlockSpec((8, _V), lambda i: (i, 0))],
        out_specs=pl.BlockSpec((8, 8), lambda i: (i, 0)),
        out_shape=jax.ShapeDtypeStruct((_B * _L, 8), jnp.float32),
    )(rows)
    zt = zcols[:, 0].reshape(_B, _L).T.reshape(_B * _L, 1)
    seq, sc = pl.pallas_call(
        _beam_body,
        out_shape=[
            jax.ShapeDtypeStruct((_B, _K * _L), jnp.int32),
            jax.ShapeDtypeStruct((_B, 8), jnp.float32),
        ],
    )(vf_t, vi_t, zt)
    tokens = seq.reshape(_B, _K, _L).transpose(0, 2, 1)
    return tokens, sc[:, :_K]


# speedup vs baseline: 1.0214x; 1.0214x over previous
import functools
import sys

import jax
import jax.numpy as jnp
from jax import lax
from jax.experimental import pallas as pl
from jax.experimental.pallas import tpu as pltpu
from jax.experimental.pallas import tpu_sc as plsc

_EPS = sys.float_info.epsilon
_V = 32768
_K = 3
_L = 16
_B = 8
_BIG_I32 = 2**30
_NEG = float("-inf")

_NC = 2
_NS = 16
_LANES = 16
_NW = _NC * _NS
_ROWS_PER_W = (_B * _L) // _NW
_NSETS = 4
_SET_ELEMS = _V // _NSETS
_SET_CHUNKS = _SET_ELEMS // _LANES
_UNROLL = 8
_NCAND = _NSETS * 0 + 3 * _LANES


def _top3_update(st, v, iv):
    m1, m2, m3, i1, i2, i3 = st
    c1 = v > m1
    nm1 = jnp.where(c1, v, m1)
    dv = jnp.where(c1, m1, v)
    ni1 = jnp.where(c1, iv, i1)
    di = jnp.where(c1, i1, iv)
    c2 = dv > m2
    nm2 = jnp.where(c2, dv, m2)
    d2v = jnp.where(c2, m2, dv)
    ni2 = jnp.where(c2, di, i2)
    d2i = jnp.where(c2, i2, di)
    c3 = d2v > m3
    nm3 = jnp.where(c3, d2v, m3)
    ni3 = jnp.where(c3, d2i, i3)
    return (nm1, nm2, nm3, ni1, ni2, ni3)


def _sc_body(x_hbm, outf_hbm, outi_hbm, buf, stage_f, stage_i, sem_a, sem_b):
    wid = lax.axis_index("s") * _NC + lax.axis_index("c")
    base_row = wid * _ROWS_PER_W
    lane = lax.iota(jnp.int32, _LANES)
    sems = [sem_a, sem_b]
    copies = [None, None]
    copies[0] = pltpu.async_copy(x_hbm.at[base_row], buf.at[0], sems[0])
    for j in range(_ROWS_PER_W):
        cur = j % 2
        nxt = (j + 1) % 2
        if j + 1 < _ROWS_PER_W:
            copies[nxt] = pltpu.async_copy(
                x_hbm.at[base_row + j + 1], buf.at[nxt], sems[nxt]
            )
        copies[cur].wait()
        bufj = buf.at[cur]

        neg = jnp.full((_LANES,), _NEG, jnp.float32)
        zero_i = jnp.zeros((_LANES,), jnp.int32)
        init = tuple((neg, neg, neg, zero_i, zero_i, zero_i) for _ in range(_NSETS))

        def sweep(i, sts, bufj=bufj, lane=lane):
            out = []
            for s in range(_NSETS):
                st = sts[s]
                for u in range(_UNROLL):
                    off = s * _SET_ELEMS + (i * _UNROLL + u) * _LANES
                    v = bufj[pl.ds(off, _LANES)]
                    iv = lane + off
                    st = _top3_update(st, v, iv)
                out.append(st)
            return tuple(out)

        sts = lax.fori_loop(0, _SET_CHUNKS // _UNROLL, sweep, init, unroll=False)

        st = sts[0]
        for s in range(1, _NSETS):
            ms1, ms2, ms3, is1, is2, is3 = sts[s]
            st = _top3_update(st, ms1, is1)
            st = _top3_update(st, ms2, is2)
            st = _top3_update(st, ms3, is3)
        m1, m2, m3, i1, i2, i3 = st

        stage_f[pl.ds(0, _LANES)] = m1
        stage_f[pl.ds(_LANES, _LANES)] = m2
        stage_f[pl.ds(2 * _LANES, _LANES)] = m3
        stage_i[pl.ds(0, _LANES)] = i1
        stage_i[pl.ds(_LANES, _LANES)] = i2
        stage_i[pl.ds(2 * _LANES, _LANES)] = i3
        r = base_row + j
        out_r = (r & (_L - 1)) * _B + (r >> 4)
        pltpu.sync_copy(stage_f, outf_hbm.at[out_r])
        pltpu.sync_copy(stage_i, outi_hbm.at[out_r])


@functools.cache
def _make_sc_rows():
    return functools.partial(
        pl.kernel,
        out_type=[
            jax.ShapeDtypeStruct((_B * _L, 3 * _LANES), jnp.float32),
            jax.ShapeDtypeStruct((_B * _L, 3 * _LANES), jnp.int32),
        ],
        mesh=plsc.VectorSubcoreMesh(
            core_axis_name="c", subcore_axis_name="s", num_cores=_NC, num_subcores=_NS
        ),
        scratch_types=[
            pltpu.VMEM((2, _V), jnp.float32),
            pltpu.VMEM((3 * _LANES,), jnp.float32),
            pltpu.VMEM((3 * _LANES,), jnp.int32),
            pltpu.SemaphoreType.DMA,
            pltpu.SemaphoreType.DMA,
        ],
    )(_sc_body)


def _sc_rows(rows):
    return _make_sc_rows()(rows)


def _z_body(x_ref, z_ref):
    x = x_ref[...]
    z = jnp.sum(jnp.exp(x), axis=1, keepdims=True)
    z_ref[...] = jnp.broadcast_to(z, (_B, 8))


def _beam_body(vf_ref, vi_ref, z_ref, seq_ref, sc_ref):
    vals = vf_ref[...]
    idx = vi_ref[...]
    lps, tks = [], []
    cur = vals
    for _ in range(_K):
        mj = jnp.max(cur, axis=1, keepdims=True)
        fj = jnp.min(jnp.where(cur == mj, idx, _BIG_I32), axis=1, keepdims=True)
        lps.append(mj)
        tks.append(fj)
        cur = jnp.where(idx == fj, _NEG, cur)
    v3_all = jnp.concatenate(lps, axis=1)
    tok_all = jnp.concatenate(tks, axis=1)
    lp_all = jnp.log(jnp.exp(v3_all) / z_ref[...] + _EPS)

    def top3(t):
        return (
            lp_all[t * _B : (t + 1) * _B, :],
            tok_all[t * _B : (t + 1) * _B, :],
        )

    scores, tok0 = top3(0)
    col = jax.lax.broadcasted_iota(jnp.int32, (_B, _L), 1)
    blocks = [jnp.where(col == 0, tok0[:, k : k + 1], 0) for k in range(_K)]
    for t in range(1, _L):
        lp, ix = top3(t)
        cand = jnp.concatenate(
            [scores[:, k : k + 1] + lp for k in range(_K)], axis=1
        )
        flat = jnp.concatenate([ix + k * _V for k in range(_K)], axis=1)
        rank = jnp.zeros((_B, 3 * _K), jnp.int32)
        for s in range(1, 3 * _K):
            rv = jnp.roll(cand, s, axis=1)
            rf = jnp.roll(flat, s, axis=1)
            beat = (rv > cand) | ((rv == cand) & (rf < flat))
            rank = rank + beat.astype(jnp.int32)
        ss, ff = [], []
        for r in range(_K):
            m = rank == r
            ss.append(jnp.sum(jnp.where(m, cand, 0.0), axis=1, keepdims=True))
            ff.append(jnp.sum(jnp.where(m, flat, 0), axis=1, keepdims=True))
        scores = jnp.concatenate(ss, axis=1)
        sel = jnp.concatenate(ff, axis=1)
        parent = sel >> 15
        token = sel & (_V - 1)
        nb = []
        for k in range(_K):
            pk = parent[:, k : k + 1]
            blk = jnp.where(pk == 0, blocks[0], jnp.where(pk == 1, blocks[1], blocks[2]))
            blk = jnp.where(col == t, token[:, k : k + 1], blk)
            nb.append(blk)
        blocks = nb
    seq_ref[...] = jnp.concatenate(blocks, axis=1)
    sc_ref[...] = jnp.concatenate(
        [scores, jnp.zeros((_B, 8 - _K), jnp.float32)], axis=1
    )


def kernel(logits):
    rows = logits.reshape(_B * _L, _V)
    vf_t, vi_t = _sc_rows(rows)
    zcols = pl.pallas_call(
        _z_body,
        grid=(_B * _L // 8,),
        in_specs=[pl.BlockSpec((8, _V), lambda i: (i, 0))],
        out_specs=pl.BlockSpec((8, 8), lambda i: (i, 0)),
        out_shape=jax.ShapeDtypeStruct((_B * _L, 8), jnp.float32),
    )(rows)
    zt = zcols[:, 0].reshape(_B, _L).T.reshape(_B * _L, 1)
    seq, sc = pl.pallas_call(
        _beam_body,
        out_shape=[
            jax.ShapeDtypeStruct((_B, _K * _L), jnp.int32),
            jax.ShapeDtypeStruct((_B, 8), jnp.float32),
        ],
    )(vf_t, vi_t, zt)
    tokens = seq.reshape(_B, _K, _L).transpose(0, 2, 1)
    return tokens, sc[:, :_K]


# speedup vs baseline: 1.0422x; 1.0204x over previous
import functools
import sys

import jax
import jax.numpy as jnp
from jax import lax
from jax.experimental import pallas as pl
from jax.experimental.pallas import tpu as pltpu
from jax.experimental.pallas import tpu_sc as plsc

_EPS = sys.float_info.epsilon
_V = 32768
_K = 3
_L = 16
_B = 8
_BIG_I32 = 2**30
_NEG = float("-inf")

_NC = 2
_NS = 16
_LANES = 16
_NW = _NC * _NS
_ROWS_PER_W = (_B * _L) // _NW
_NSETS = 4
_SET_ELEMS = _V // _NSETS
_SET_CHUNKS = _SET_ELEMS // _LANES
_UNROLL = 8
_NCAND = _NSETS * 0 + 3 * _LANES


def _top3_update(st, v, iv):
    m1, m2, m3, i1, i2, i3 = st
    c1 = v > m1
    nm1 = jnp.where(c1, v, m1)
    dv = jnp.where(c1, m1, v)
    ni1 = jnp.where(c1, iv, i1)
    di = jnp.where(c1, i1, iv)
    c2 = dv > m2
    nm2 = jnp.where(c2, dv, m2)
    d2v = jnp.where(c2, m2, dv)
    ni2 = jnp.where(c2, di, i2)
    d2i = jnp.where(c2, i2, di)
    c3 = d2v > m3
    nm3 = jnp.where(c3, d2v, m3)
    ni3 = jnp.where(c3, d2i, i3)
    return (nm1, nm2, nm3, ni1, ni2, ni3)


def _sc_body(x_hbm, outf_hbm, outi_hbm, buf, stage_f, stage_i, sem_a, sem_b):
    wid = lax.axis_index("s") * _NC + lax.axis_index("c")
    base_row = wid * _ROWS_PER_W
    lane = lax.iota(jnp.int32, _LANES)
    sems = [sem_a, sem_b]
    copies = [None, None]
    copies[0] = pltpu.async_copy(x_hbm.at[base_row], buf.at[0], sems[0])
    for j in range(_ROWS_PER_W):
        cur = j % 2
        nxt = (j + 1) % 2
        if j + 1 < _ROWS_PER_W:
            copies[nxt] = pltpu.async_copy(
                x_hbm.at[base_row + j + 1], buf.at[nxt], sems[nxt]
            )
        copies[cur].wait()
        bufj = buf.at[cur]

        neg = jnp.full((_LANES,), _NEG, jnp.float32)
        zero_i = jnp.zeros((_LANES,), jnp.int32)
        init = tuple((neg, neg, neg, zero_i, zero_i, zero_i) for _ in range(_NSETS))

        def sweep(i, sts, bufj=bufj, lane=lane):
            out = []
            for s in range(_NSETS):
                st = sts[s]
                for u in range(_UNROLL):
                    off = s * _SET_ELEMS + (i * _UNROLL + u) * _LANES
                    v = bufj[pl.ds(off, _LANES)]
                    iv = lane + off
                    st = _top3_update(st, v, iv)
                out.append(st)
            return tuple(out)

        sts = lax.fori_loop(0, _SET_CHUNKS // _UNROLL, sweep, init, unroll=False)

        st = sts[0]
        for s in range(1, _NSETS):
            ms1, ms2, ms3, is1, is2, is3 = sts[s]
            st = _top3_update(st, ms1, is1)
            st = _top3_update(st, ms2, is2)
            st = _top3_update(st, ms3, is3)
        m1, m2, m3, i1, i2, i3 = st

        stage_f[pl.ds(0, _LANES)] = m1
        stage_f[pl.ds(_LANES, _LANES)] = m2
        stage_f[pl.ds(2 * _LANES, _LANES)] = m3
        stage_i[pl.ds(0, _LANES)] = i1
        stage_i[pl.ds(_LANES, _LANES)] = i2
        stage_i[pl.ds(2 * _LANES, _LANES)] = i3
        r = base_row + j
        out_r = (r & (_L - 1)) * _B + (r >> 4)
        pltpu.sync_copy(stage_f, outf_hbm.at[out_r])
        pltpu.sync_copy(stage_i, outi_hbm.at[out_r])


@functools.cache
def _make_sc_rows():
    return functools.partial(
        pl.kernel,
        out_type=[
            jax.ShapeDtypeStruct((_B * _L, 3 * _LANES), jnp.float32),
            jax.ShapeDtypeStruct((_B * _L, 3 * _LANES), jnp.int32),
        ],
        mesh=plsc.VectorSubcoreMesh(
            core_axis_name="c", subcore_axis_name="s", num_cores=_NC, num_subcores=_NS
        ),
        scratch_types=[
            pltpu.VMEM((2, _V), jnp.float32),
            pltpu.VMEM((3 * _LANES,), jnp.float32),
            pltpu.VMEM((3 * _LANES,), jnp.int32),
            pltpu.SemaphoreType.DMA,
            pltpu.SemaphoreType.DMA,
        ],
    )(_sc_body)


def _sc_rows(rows):
    return _make_sc_rows()(rows)


def _z_body(x_ref, z_ref):
    x = x_ref[...]
    z = jnp.sum(jnp.exp(x), axis=1, keepdims=True)
    z_ref[...] = jnp.broadcast_to(z, (_B, 8))


def _beam_body(vf_ref, vi_ref, z_ref, seq_ref, sc_ref):
    vals = vf_ref[...]
    idx = vi_ref[...]
    lps, tks = [], []
    cur = vals
    for _ in range(_K):
        mj = jnp.max(cur, axis=1, keepdims=True)
        fj = jnp.min(jnp.where(cur == mj, idx, _BIG_I32), axis=1, keepdims=True)
        lps.append(mj)
        tks.append(fj)
        cur = jnp.where(idx == fj, _NEG, cur)
    v3_all = jnp.concatenate(lps, axis=1)
    tok_all = jnp.concatenate(tks, axis=1)
    lp_all = jnp.log(jnp.exp(v3_all) / z_ref[...] + _EPS)

    def top3(t):
        return (
            lp_all[t * _B : (t + 1) * _B, :],
            tok_all[t * _B : (t + 1) * _B, :],
        )

    scores, tok0 = top3(0)
    col = jax.lax.broadcasted_iota(jnp.int32, (_B, _L), 1)
    blocks = [jnp.where(col == 0, tok0[:, k : k + 1], 0) for k in range(_K)]
    for t in range(1, _L):
        lp, ix = top3(t)
        s0, s1, s2 = scores[:, 0:1], scores[:, 1:2], scores[:, 2:3]
        l0, l1, l2 = lp[:, 0:1], lp[:, 1:2], lp[:, 2:3]
        x0, x1, x2 = ix[:, 0:1], ix[:, 1:2], ix[:, 2:3]
        c00 = s0 + l0
        c01 = s0 + l1
        c10 = s1 + l0
        c02 = s0 + l2
        c11 = s1 + l1
        c20 = s2 + l0
        b2 = c01 >= c10
        v2 = jnp.where(b2, c01, c10)
        p2 = jnp.where(b2, 0, 1)
        k2 = jnp.where(b2, x1, x0)
        ba = c02 >= c10
        va = jnp.where(ba, c02, c10)
        pa = jnp.where(ba, 0, 1)
        ka = jnp.where(ba, x2, x0)
        vb, pb, kb = c01, jnp.zeros_like(p2), x1
        m1_ = c11 > vb
        vb = jnp.where(m1_, c11, vb)
        pb = jnp.where(m1_, 1, pb)
        kb = jnp.where(m1_, x1, kb)
        m2_ = c20 > vb
        vb = jnp.where(m2_, c20, vb)
        pb = jnp.where(m2_, 2, pb)
        kb = jnp.where(m2_, x0, kb)
        v3 = jnp.where(b2, va, vb)
        p3 = jnp.where(b2, pa, pb)
        k3 = jnp.where(b2, ka, kb)
        scores = jnp.concatenate([c00, v2, v3], axis=1)
        parent = jnp.concatenate([jnp.zeros_like(p2), p2, p3], axis=1)
        token = jnp.concatenate([x0, k2, k3], axis=1)
        nb = []
        for k in range(_K):
            pk = parent[:, k : k + 1]
            blk = jnp.where(pk == 0, blocks[0], jnp.where(pk == 1, blocks[1], blocks[2]))
            blk = jnp.where(col == t, token[:, k : k + 1], blk)
            nb.append(blk)
        blocks = nb
    seq_ref[...] = jnp.concatenate(blocks, axis=1)
    sc_ref[...] = jnp.concatenate(
        [scores, jnp.zeros((_B, 8 - _K), jnp.float32)], axis=1
    )


def kernel(logits):
    rows = logits.reshape(_B * _L, _V)
    vf_t, vi_t = _sc_rows(rows)
    zcols = pl.pallas_call(
        _z_body,
        grid=(_B * _L // 8,),
        in_specs=[pl.BlockSpec((8, _V), lambda i: (i, 0))],
        out_specs=pl.BlockSpec((8, 8), lambda i: (i, 0)),
        out_shape=jax.ShapeDtypeStruct((_B * _L, 8), jnp.float32),
    )(rows)
    zt = zcols[:, 0].reshape(_B, _L).T.reshape(_B * _L, 1)
    seq, sc = pl.pallas_call(
        _beam_body,
        out_shape=[
            jax.ShapeDtypeStruct((_B, _K * _L), jnp.int32),
            jax.ShapeDtypeStruct((_B, 8), jnp.float32),
        ],
    )(vf_t, vi_t, zt)
    tokens = seq.reshape(_B, _K, _L).transpose(0, 2, 1)
    return tokens, sc[:, :_K]
